# baseline (speedup 1.0000x reference)
import functools

import jax
import jax.numpy as jnp
from jax import lax
from jax.experimental import pallas as pl
from jax.experimental.pallas import tpu as pltpu
from jax.experimental.pallas import tpu_sc as plsc

NR = 8192
NZ = 2048
N_QUERY = 1000000

NC = 2
NS = 16
NW = NC * NS
L = 16

C = 2048
GW = 128
PW = 32768
NPAD = NW * PW


def _make_kernel():
    mesh = plsc.VectorSubcoreMesh(core_axis_name="c", subcore_axis_name="s")

    @functools.partial(
        pl.kernel,
        mesh=mesh,
        out_type=jax.ShapeDtypeStruct((NPAD,), jnp.float32),
        scratch_types=[
            pltpu.VMEM((C,), jnp.float32),
            pltpu.VMEM((C,), jnp.float32),
            pltpu.VMEM((C,), jnp.int32),
            pltpu.VMEM((C,), jnp.int32),
            pltpu.VMEM((C,), jnp.int32),
            pltpu.VMEM((C,), jnp.int32),
            pltpu.VMEM((C,), jnp.float32),
            pltpu.VMEM((C,), jnp.float32),
            pltpu.VMEM((C,), jnp.float32),
            pltpu.VMEM((C,), jnp.float32),
            pltpu.VMEM((C,), jnp.float32),
            pltpu.VMEM((C,), jnp.float32),
            pltpu.VMEM((C,), jnp.float32),
            pltpu.SemaphoreType.DMA,
            pltpu.SemaphoreType.DMA,
        ],
    )
    def k(r_hbm, z_hbm, tab_hbm, out_hbm,
          r_v, z_v, i00, i01, i10, i11, t00, t01, t10, t11,
          wr_v, wz_v, o_v, sem_in, sem_g):
        wid = lax.axis_index("s") * NC + lax.axis_index("c")
        base = wid * PW

        def chunk_body(ci, carry):
            off = base + ci * C
            cp_r = pltpu.async_copy(r_hbm.at[pl.ds(off, C)], r_v, sem_in)
            cp_z = pltpu.async_copy(z_hbm.at[pl.ds(off, C)], z_v, sem_in)
            cp_r.wait()
            cp_z.wait()

            def idx_body(i, carry2):
                s = pl.ds(i * L, L)
                rr = r_v[s]
                zz = z_v[s]
                ir0 = jnp.minimum(jnp.maximum(rr.astype(jnp.int32), 0), NR - 2)
                iz0 = jnp.minimum(jnp.maximum(zz.astype(jnp.int32), 0), NZ - 2)
                wr = jnp.clip(rr - ir0.astype(jnp.float32), 0.0, 1.0)
                wz = jnp.clip(zz - iz0.astype(jnp.float32), 0.0, 1.0)
                b = ir0 * NZ + iz0
                i00[s] = b
                i01[s] = b + 1
                i10[s] = b + NZ
                i11[s] = b + NZ + 1
                wr_v[s] = wr
                wz_v[s] = wz
                return carry2

            lax.fori_loop(0, C // L, idx_body, 0)

            cps = []
            for j in range(C // GW):
                s = pl.ds(j * GW, GW)
                cps.append(pltpu.async_copy(tab_hbm.at[i00.at[s]], t00.at[s], sem_g))
                cps.append(pltpu.async_copy(tab_hbm.at[i01.at[s]], t01.at[s], sem_g))
                cps.append(pltpu.async_copy(tab_hbm.at[i10.at[s]], t10.at[s], sem_g))
                cps.append(pltpu.async_copy(tab_hbm.at[i11.at[s]], t11.at[s], sem_g))
            for cp in cps:
                cp.wait()

            def comb_body(i, carry2):
                s = pl.ds(i * L, L)
                wr = wr_v[s]
                wz = wz_v[s]
                a = t00[s] * (1.0 - wr) + t10[s] * wr
                b2 = t01[s] * (1.0 - wr) + t11[s] * wr
                o_v[s] = a * (1.0 - wz) + b2 * wz
                return carry2

            lax.fori_loop(0, C // L, comb_body, 0)

            pltpu.sync_copy(o_v, out_hbm.at[pl.ds(off, C)])
            return carry

        lax.fori_loop(0, PW // C, chunk_body, 0)

    return k


_sc_interp = _make_kernel()


def kernel(r, z, timetable):
    pad = NPAD - N_QUERY
    r_p = jnp.pad(r, (0, pad))
    z_p = jnp.pad(z, (0, pad))
    tab_flat = timetable.reshape(NR * NZ)
    out = _sc_interp(r_p, z_p, tab_flat)
    return out[:N_QUERY]


# speedup vs baseline: 2.3673x; 2.3673x over previous
import functools

import jax
import jax.numpy as jnp
from jax import lax
from jax.experimental import pallas as pl
from jax.experimental.pallas import tpu as pltpu
from jax.experimental.pallas import tpu_sc as plsc

NR = 8192
NZ = 2048
N_QUERY = 1000000

NC = 2
NS = 16
NW = NC * NS
L = 16

C = 1024
GW = 128
NCHUNK = (N_QUERY + C - 1) // C
K = (NCHUNK + NW - 1) // NW


def _make_kernel():
    mesh = plsc.VectorSubcoreMesh(core_axis_name="c", subcore_axis_name="s")

    @functools.partial(
        pl.kernel,
        mesh=mesh,
        out_type=jax.ShapeDtypeStruct((N_QUERY,), jnp.float32),
        scratch_types=[
            pltpu.VMEM((C,), jnp.float32),
            pltpu.VMEM((C,), jnp.float32),
            pltpu.VMEM((C,), jnp.int32),
            pltpu.VMEM((C,), jnp.int32),
            pltpu.VMEM((C,), jnp.int32),
            pltpu.VMEM((C,), jnp.int32),
            pltpu.VMEM((C,), jnp.float32),
            pltpu.VMEM((C,), jnp.float32),
            pltpu.VMEM((C,), jnp.float32),
            pltpu.VMEM((C,), jnp.float32),
            pltpu.VMEM((C,), jnp.float32),
            pltpu.VMEM((C,), jnp.float32),
            pltpu.VMEM((C,), jnp.float32),
            pltpu.SemaphoreType.DMA,
            pltpu.SemaphoreType.DMA,
        ],
    )
    def k(r_hbm, z_hbm, tab_hbm, out_hbm,
          r_v, z_v, i00, i01, i10, i11, t00, t01, t10, t11,
          wr_v, wz_v, o_v, sem_in, sem_g):
        wid = lax.axis_index("s") * NC + lax.axis_index("c")

        def chunk_body(ci, carry):
            cid = wid + ci * NW
            off = jnp.minimum(cid * C, N_QUERY - C)
            cp_r = pltpu.async_copy(r_hbm.at[pl.ds(off, C)], r_v, sem_in)
            cp_z = pltpu.async_copy(z_hbm.at[pl.ds(off, C)], z_v, sem_in)
            cp_r.wait()
            cp_z.wait()

            def idx_body(i, carry2):
                s = pl.ds(i * L, L)
                rr = r_v[s]
                zz = z_v[s]
                ir0 = jnp.minimum(jnp.maximum(rr.astype(jnp.int32), 0), NR - 2)
                iz0 = jnp.minimum(jnp.maximum(zz.astype(jnp.int32), 0), NZ - 2)
                wr = jnp.clip(rr - ir0.astype(jnp.float32), 0.0, 1.0)
                wz = jnp.clip(zz - iz0.astype(jnp.float32), 0.0, 1.0)
                b = ir0 * NZ + iz0
                i00[s] = b
                i01[s] = b + 1
                i10[s] = b + NZ
                i11[s] = b + NZ + 1
                wr_v[s] = wr
                wz_v[s] = wz
                return carry2

            lax.fori_loop(0, C // L, idx_body, 0)

            cps = []
            for j in range(C // GW):
                s = pl.ds(j * GW, GW)
                cps.append(pltpu.async_copy(tab_hbm.at[i00.at[s]], t00.at[s], sem_g))
                cps.append(pltpu.async_copy(tab_hbm.at[i01.at[s]], t01.at[s], sem_g))
                cps.append(pltpu.async_copy(tab_hbm.at[i10.at[s]], t10.at[s], sem_g))
                cps.append(pltpu.async_copy(tab_hbm.at[i11.at[s]], t11.at[s], sem_g))
            for cp in cps:
                cp.wait()

            def comb_body(i, carry2):
                s = pl.ds(i * L, L)
                wr = wr_v[s]
                wz = wz_v[s]
                a = t00[s] * (1.0 - wr) + t10[s] * wr
                b2 = t01[s] * (1.0 - wr) + t11[s] * wr
                o_v[s] = a * (1.0 - wz) + b2 * wz
                return carry2

            lax.fori_loop(0, C // L, comb_body, 0)

            pltpu.sync_copy(o_v, out_hbm.at[pl.ds(off, C)])
            return carry

        lax.fori_loop(0, K, chunk_body, 0)

    return k


_sc_interp = _make_kernel()


def kernel(r, z, timetable):
    tab_flat = timetable.reshape(NR * NZ)
    return _sc_interp(r, z, tab_flat)


# speedup vs baseline: 2.9786x; 1.2582x over previous
import functools

import jax
import jax.numpy as jnp
from jax import lax
from jax.experimental import pallas as pl
from jax.experimental.pallas import tpu as pltpu
from jax.experimental.pallas import tpu_sc as plsc

NR = 8192
NZ = 2048
N_QUERY = 1000000

NC = 2
NS = 16
NW = NC * NS
L = 16

C = 1024
GW = 128
NCHUNK = (N_QUERY + C - 1) // C
K = (NCHUNK + NW - 1) // NW
H = (K - 1) // 2


def _make_kernel():
    mesh = plsc.VectorSubcoreMesh(core_axis_name="c", subcore_axis_name="s")

    set_scratch = [
        pltpu.VMEM((C,), jnp.float32),
        pltpu.VMEM((C,), jnp.float32),
        pltpu.VMEM((4 * C,), jnp.int32),
        pltpu.VMEM((4 * C,), jnp.float32),
        pltpu.VMEM((C,), jnp.float32),
        pltpu.VMEM((C,), jnp.float32),
        pltpu.VMEM((C,), jnp.float32),
        pltpu.SemaphoreType.DMA,
        pltpu.SemaphoreType.DMA,
    ]

    @functools.partial(
        pl.kernel,
        mesh=mesh,
        out_type=jax.ShapeDtypeStruct((N_QUERY,), jnp.float32),
        scratch_types=set_scratch + set_scratch,
    )
    def k(r_hbm, z_hbm, tab_hbm, out_hbm,
          rA, zA, iA, tA, wrA, wzA, oA, semzA, semgA,
          rB, zB, iB, tB, wrB, wzB, oB, semzB, semgB):
        wid = lax.axis_index("s") * NC + lax.axis_index("c")
        A = (rA, zA, iA, tA, wrA, wzA, oA, semzA, semgA)
        B = (rB, zB, iB, tB, wrB, wzB, oB, semzB, semgB)

        def slot_off(slot):
            cid = wid + slot * NW
            return jnp.minimum(cid * C, N_QUERY - C)

        def start_rz(bufs, slot):
            r_v, z_v = bufs[0], bufs[1]
            sem = bufs[7]
            off = slot_off(slot)
            pltpu.async_copy(r_hbm.at[pl.ds(off, C)], r_v, sem)
            pltpu.async_copy(z_hbm.at[pl.ds(off, C)], z_v, sem)

        def drain_rz(bufs):
            r_v, z_v = bufs[0], bufs[1]
            sem = bufs[7]
            pltpu.make_async_copy(r_hbm.at[pl.ds(0, C)], r_v, sem).wait()
            pltpu.make_async_copy(z_hbm.at[pl.ds(0, C)], z_v, sem).wait()

        def fire_slot(bufs):
            r_v, z_v, i_v, t_v = bufs[0], bufs[1], bufs[2], bufs[3]
            wr_v, wz_v = bufs[4], bufs[5]
            sem_g = bufs[8]
            drain_rz(bufs)

            def idx_body(i, carry):
                s = pl.ds(i * L, L)
                rr = r_v[s]
                zz = z_v[s]
                ir0 = jnp.minimum(jnp.maximum(rr.astype(jnp.int32), 0), NR - 2)
                iz0 = jnp.minimum(jnp.maximum(zz.astype(jnp.int32), 0), NZ - 2)
                wr = jnp.clip(rr - ir0.astype(jnp.float32), 0.0, 1.0)
                wz = jnp.clip(zz - iz0.astype(jnp.float32), 0.0, 1.0)
                b = ir0 * NZ + iz0
                i_v[pl.ds(0 * C + i * L, L)] = b
                i_v[pl.ds(1 * C + i * L, L)] = b + 1
                i_v[pl.ds(2 * C + i * L, L)] = b + NZ
                i_v[pl.ds(3 * C + i * L, L)] = b + NZ + 1
                wr_v[s] = wr
                wz_v[s] = wz
                return carry

            lax.fori_loop(0, C // L, idx_body, 0)

            for q in range(4):
                for j in range(C // GW):
                    s = pl.ds(q * C + j * GW, GW)
                    pltpu.async_copy(tab_hbm.at[i_v.at[s]], t_v.at[s], sem_g)

        def finish_slot(bufs, slot):
            t_v, wr_v, wz_v, o_v = bufs[3], bufs[4], bufs[5], bufs[6]
            sem_g = bufs[8]
            pltpu.make_async_copy(tab_hbm.at[pl.ds(0, 4 * C)], t_v, sem_g).wait()

            def comb_body(i, carry):
                s = pl.ds(i * L, L)
                wr = wr_v[s]
                wz = wz_v[s]
                t00 = t_v[pl.ds(0 * C + i * L, L)]
                t01 = t_v[pl.ds(1 * C + i * L, L)]
                t10 = t_v[pl.ds(2 * C + i * L, L)]
                t11 = t_v[pl.ds(3 * C + i * L, L)]
                a = t00 * (1.0 - wr) + t10 * wr
                b2 = t01 * (1.0 - wr) + t11 * wr
                o_v[s] = a * (1.0 - wz) + b2 * wz
                return carry

            lax.fori_loop(0, C // L, comb_body, 0)
            pltpu.sync_copy(o_v, out_hbm.at[pl.ds(slot_off(slot), C)])

        start_rz(A, 0)
        fire_slot(A)
        start_rz(B, 1)

        def pair_body(h, carry):
            fire_slot(B)
            start_rz(A, 2 * h + 2)
            finish_slot(A, 2 * h)
            fire_slot(A)
            start_rz(B, 2 * h + 3)
            finish_slot(B, 2 * h + 1)
            return carry

        lax.fori_loop(0, H, pair_body, 0)
        finish_slot(A, 2 * H)
        drain_rz(B)

    return k


_sc_interp = _make_kernel()


def kernel(r, z, timetable):
    tab_flat = timetable.reshape(NR * NZ)
    return _sc_interp(r, z, tab_flat)
